# Initial kernel scaffold; baseline (speedup 1.0000x reference)
import jax
import jax.numpy as jnp
from jax.experimental import pallas as pl


def kernel(p_query, p_context, h_context, W_raise, b_raise, W_pw0, b_pw0, g_pw0, be_pw0, W_pw1, b_pw1, g_pw1, be_pw1, W_pw2, b_pw2, g_pw2, be_pw2, W_g0, b_g0, g_g0, be_g0, W_g1, b_g1, g_g1, be_g1, W_g2, b_g2):
    raise NotImplementedError("write your pallas kernel here")


# speedup vs baseline: 4.9177x; 4.9177x over previous
import functools
import math

import jax
import jax.numpy as jnp
from jax import lax
from jax.experimental import pallas as pl
from jax.experimental.pallas import tpu as pltpu
from jax.experimental.pallas import tpu_sc as plsc

RADIUS = 0.5
K = 32
B_, NQ, NC, H = 4, 1024, 8192, 128
TQ = 128
RT = 2048
QT = 64
NROW = B_ * NQ * K
NQF = B_ * NQ


def _knn_body(qp_ref, ct_ref, idx_ref, c_ref):
    t = pl.program_id(0)
    b = t // (NQ // TQ)
    qp = qp_ref[...]
    ct = ct_ref[0]
    q2 = jnp.sum(qp * qp, axis=1, keepdims=True)
    c2 = jnp.sum(ct * ct, axis=0, keepdims=True)
    qc = jax.lax.dot_general(qp, ct, (((1,), (0,)), ((), ())),
                             preferred_element_type=jnp.float32)
    D = jnp.maximum(q2 + c2 - 2.0 * qc, 0.0)

    lane = jax.lax.broadcasted_iota(jnp.int32, (TQ, NC), 1)
    col = jax.lax.broadcasted_iota(jnp.int32, (TQ, K), 1)

    def step(i, carry):
        D, acc_i, acc_d = carry
        m = jnp.min(D, axis=1, keepdims=True)
        am = jnp.min(jnp.where(D == m, lane, NC), axis=1, keepdims=True)
        acc_i = jnp.where(col == i, am, acc_i)
        acc_d = jnp.where(col == i, m, acc_d)
        D = jnp.where(lane == am, jnp.float32(1e30), D)
        return D, acc_i, acc_d

    acc_i0 = jnp.zeros((TQ, K), jnp.int32)
    acc_d0 = jnp.zeros((TQ, K), jnp.float32)
    _, acc_i, acc_d = lax.fori_loop(0, K, step, (D, acc_i0, acc_d0))

    keep = acc_d <= RADIUS ** 2
    idx_local = jnp.where(keep, acc_i, 0)
    d = jnp.where(keep, acc_d, 0.0)
    idx_ref[...] = idx_local + b * NC
    cw = 0.5 * (jnp.cos(d * (math.pi / RADIUS)) + 1.0)
    c_ref[...] = jnp.where((d > 0.0) & keep, cw, 0.0)


def _knn(qp, ct):
    grid = (NQF // TQ,)
    return pl.pallas_call(
        _knn_body,
        grid=grid,
        in_specs=[
            pl.BlockSpec((TQ, 8), lambda t: (t, 0)),
            pl.BlockSpec((1, 8, NC), lambda t: (t // (NQ // TQ), 0, 0)),
        ],
        out_specs=[
            pl.BlockSpec((TQ, K), lambda t: (t, 0)),
            pl.BlockSpec((TQ, K), lambda t: (t, 0)),
        ],
        out_shape=[
            jax.ShapeDtypeStruct((NQF, K), jnp.int32),
            jax.ShapeDtypeStruct((NQF, K), jnp.float32),
        ],
        compiler_params=pltpu.CompilerParams(
            dimension_semantics=("arbitrary",)),
    )(qp, ct)


_CHUNK = 128


def _sc_gather(h_tab, p_tab, idx_flat):
    info = plsc.get_sparse_core_info()
    nw = info.num_cores * info.num_subcores
    rows_w = NROW // nw
    nch = rows_w // _CHUNK
    mesh = plsc.VectorSubcoreMesh(core_axis_name="c", subcore_axis_name="s")

    @functools.partial(
        pl.kernel, mesh=mesh,
        out_type=[
            jax.ShapeDtypeStruct((NROW, H), jnp.float32),
            jax.ShapeDtypeStruct((NROW, 128), jnp.float32),
        ],
        scratch_types=[
            pltpu.VMEM((_CHUNK,), jnp.int32),
            pltpu.VMEM((_CHUNK, H), jnp.float32),
            pltpu.VMEM((_CHUNK, 128), jnp.float32),
            pltpu.SemaphoreType.DMA,
        ],
    )
    def kern(h_hbm, p_hbm, idx_hbm, oh_hbm, op_hbm, idx_v, h_v, p_v, sem):
        wid = lax.axis_index("s") * info.num_cores + lax.axis_index("c")
        base = wid * rows_w

        def body(g, carry):
            off = base + g * _CHUNK
            pltpu.sync_copy(idx_hbm.at[pl.ds(off, _CHUNK)], idx_v)
            pltpu.async_copy(h_hbm.at[idx_v], h_v, sem).wait()
            pltpu.async_copy(p_hbm.at[idx_v], p_v, sem).wait()
            pltpu.sync_copy(h_v, oh_hbm.at[pl.ds(off, _CHUNK)])
            pltpu.sync_copy(p_v, op_hbm.at[pl.ds(off, _CHUNK)])
            return carry

        lax.fori_loop(0, nch, body, 0)

    return kern(h_tab, p_tab, idx_flat)


def _mlp0_body(hg_ref, pg_ref, qr_ref, wr_ref, br_ref, w0a_ref, w0b_ref,
               b0_ref, z_ref, st_ref):
    t = pl.program_id(0)
    prel = pg_ref[...][:, :16] - qr_ref[...]
    hrel = jnp.dot(prel, wr_ref[...],
                   preferred_element_type=jnp.float32) + br_ref[...]
    z = (jnp.dot(hrel, w0a_ref[...], preferred_element_type=jnp.float32)
         + jnp.dot(hg_ref[...], w0b_ref[...],
                   preferred_element_type=jnp.float32) + b0_ref[...])
    z_ref[...] = z

    @pl.when(t == 0)
    def _():
        st_ref[...] = jnp.zeros_like(st_ref)

    s1 = jnp.sum(z, axis=0, keepdims=True)
    s2 = jnp.sum(z * z, axis=0, keepdims=True)
    st_ref[...] += jnp.concatenate(
        [s1, s2, jnp.zeros((6, z.shape[1]), jnp.float32)], axis=0)


def _mlp_body(z_ref, st_in_ref, g_ref, be_ref, w_ref, b_ref, z_out_ref,
              st_ref, *, n):
    t = pl.program_id(0)
    st = st_in_ref[...]
    mean = st[0:1, :] / n
    var = st[1:2, :] / n - mean * mean
    y = (z_ref[...] - mean) * jax.lax.rsqrt(var + 1e-5) * g_ref[...] \
        + be_ref[...]
    y = jnp.maximum(y, 0.0)
    z = jnp.dot(y, w_ref[...], preferred_element_type=jnp.float32) + b_ref[...]
    z_out_ref[...] = z

    @pl.when(t == 0)
    def _():
        st_ref[...] = jnp.zeros_like(st_ref)

    s1 = jnp.sum(z, axis=0, keepdims=True)
    s2 = jnp.sum(z * z, axis=0, keepdims=True)
    st_ref[...] += jnp.concatenate(
        [s1, s2, jnp.zeros((6, z.shape[1]), jnp.float32)], axis=0)


def _full(shape):
    return pl.BlockSpec(shape, lambda *_: tuple(0 for _ in shape))


def _mlp0(h_g, p_g, q_rep, wr, br, w0a, w0b, b0):
    grid = (NROW // RT,)
    row = pl.BlockSpec((RT, None), lambda t: (t, 0))
    return pl.pallas_call(
        _mlp0_body,
        grid=grid,
        in_specs=[
            pl.BlockSpec((RT, H), lambda t: (t, 0)),
            pl.BlockSpec((RT, 128), lambda t: (t, 0)),
            pl.BlockSpec((RT, 16), lambda t: (t, 0)),
            _full((16, 64)), _full((1, 64)),
            _full((64, 128)), _full((128, 128)), _full((1, 128)),
        ],
        out_specs=[
            pl.BlockSpec((RT, 128), lambda t: (t, 0)),
            _full((8, 128)),
        ],
        out_shape=[
            jax.ShapeDtypeStruct((NROW, 128), jnp.float32),
            jax.ShapeDtypeStruct((8, 128), jnp.float32),
        ],
        compiler_params=pltpu.CompilerParams(
            dimension_semantics=("arbitrary",)),
    )(h_g, p_g, q_rep, wr, br, w0a, w0b, b0)


def _mlp_layer(z, st_in, g, be, w, b, cout):
    grid = (NROW // RT,)
    cin = z.shape[1]
    return pl.pallas_call(
        functools.partial(_mlp_body, n=float(NROW)),
        grid=grid,
        in_specs=[
            pl.BlockSpec((RT, cin), lambda t: (t, 0)),
            _full((8, cin)),
            _full((1, cin)), _full((1, cin)),
            _full((cin, cout)), _full((1, cout)),
        ],
        out_specs=[
            pl.BlockSpec((RT, cout), lambda t: (t, 0)),
            _full((8, cout)),
        ],
        out_shape=[
            jax.ShapeDtypeStruct((NROW, cout), jnp.float32),
            jax.ShapeDtypeStruct((8, cout), jnp.float32),
        ],
        compiler_params=pltpu.CompilerParams(
            dimension_semantics=("arbitrary",)),
    )(z, st_in, g, be, w, b)


def _ksum_body(z_ref, st_ref, g_ref, be_ref, c_ref, s_ref):
    st = st_ref[...]
    n = float(NROW)
    mean = (st[0:1, :] / n).reshape(1, 1, 256)
    var = (st[1:2, :] / n).reshape(1, 1, 256) - mean * mean
    g = g_ref[...].reshape(1, 1, 256)
    be = be_ref[...].reshape(1, 1, 256)
    y = (z_ref[...] - mean) * jax.lax.rsqrt(var + 1e-5) * g + be
    y = y * c_ref[...][:, :, None]
    s_ref[...] = jnp.sum(y, axis=1)


def _ksum(z2r, st2, g, be, cwin):
    grid = (NQF // QT,)
    return pl.pallas_call(
        _ksum_body,
        grid=grid,
        in_specs=[
            pl.BlockSpec((QT, K, 256), lambda t: (t, 0, 0)),
            _full((8, 256)),
            _full((1, 256)), _full((1, 256)),
            pl.BlockSpec((QT, K), lambda t: (t, 0)),
        ],
        out_specs=pl.BlockSpec((QT, 256), lambda t: (t, 0)),
        out_shape=jax.ShapeDtypeStruct((NQF, 256), jnp.float32),
        compiler_params=pltpu.CompilerParams(
            dimension_semantics=("arbitrary",)),
    )(z2r, st2, g, be, cwin)


def _head_body(s_ref, w0_ref, b0_ref, g0_ref, be0_ref, w1_ref, b1_ref,
               g1_ref, be1_ref, w2_ref, b2_ref, o_ref):
    n = float(NQF)

    def bn(z, g, be):
        m = jnp.sum(z, axis=0, keepdims=True) / n
        v = jnp.sum(z * z, axis=0, keepdims=True) / n - m * m
        return (z - m) * jax.lax.rsqrt(v + 1e-5) * g + be

    z = jnp.dot(s_ref[...], w0_ref[...],
                preferred_element_type=jnp.float32) + b0_ref[...]
    y = jnp.maximum(bn(z, g0_ref[...], be0_ref[...]), 0.0)
    z = jnp.dot(y, w1_ref[...], preferred_element_type=jnp.float32) \
        + b1_ref[...]
    y = jnp.maximum(bn(z, g1_ref[...], be1_ref[...]), 0.0)
    o_ref[...] = jnp.dot(y, w2_ref[...],
                         preferred_element_type=jnp.float32) + b2_ref[...]


def _head(s, w0, b0, g0, be0, w1, b1, g1, be1, w2p, b2p):
    return pl.pallas_call(
        _head_body,
        in_specs=[
            _full((NQF, 256)),
            _full((256, 128)), _full((1, 128)), _full((1, 128)),
            _full((1, 128)),
            _full((128, 64)), _full((1, 64)), _full((1, 64)), _full((1, 64)),
            _full((64, 8)), _full((1, 8)),
        ],
        out_specs=_full((NQF, 8)),
        out_shape=jax.ShapeDtypeStruct((NQF, 8), jnp.float32),
    )(s, w0, b0, g0, be0, w1, b1, g1, be1, w2p, b2p)


def kernel(p_query, p_context, h_context, W_raise, b_raise,
           W_pw0, b_pw0, g_pw0, be_pw0,
           W_pw1, b_pw1, g_pw1, be_pw1,
           W_pw2, b_pw2, g_pw2, be_pw2,
           W_g0, b_g0, g_g0, be_g0,
           W_g1, b_g1, g_g1, be_g1,
           W_g2, b_g2):
    qp = jnp.pad(p_query.reshape(NQF, 3), ((0, 0), (0, 5)))
    ct = jnp.pad(jnp.swapaxes(p_context, 1, 2), ((0, 0), (0, 5), (0, 0)))
    h_tab = h_context.reshape(B_ * NC, H)
    p_tab = jnp.pad(p_context.reshape(B_ * NC, 3), ((0, 0), (0, 125)))
    q_rep = jnp.pad(
        jnp.broadcast_to(p_query.reshape(NQF, 1, 3), (NQF, K, 3)
                         ).reshape(NROW, 3), ((0, 0), (0, 13)))
    wr = jnp.pad(W_raise.T, ((0, 13), (0, 0)))
    w0a, w0b = W_pw0[:64], W_pw0[64:]
    r1 = lambda v: v.reshape(1, -1)
    w2p = jnp.pad(W_g2, ((0, 0), (0, 5)))
    b2p = jnp.pad(b_g2, (0, 5)).reshape(1, 8)

    idx_flat, cwin = _knn(qp, ct)

    h_g, p_g = _sc_gather(h_tab, p_tab, idx_flat.reshape(NROW))

    z0, st0 = _mlp0(h_g, p_g, q_rep, wr, r1(b_raise), w0a, w0b, r1(b_pw0))
    z1, st1 = _mlp_layer(z0, st0, r1(g_pw0), r1(be_pw0), W_pw1, r1(b_pw1), 128)
    z2, st2 = _mlp_layer(z1, st1, r1(g_pw1), r1(be_pw1), W_pw2, r1(b_pw2), 256)
    s = _ksum(z2.reshape(NQF, K, 256), st2, r1(g_pw2), r1(be_pw2), cwin)

    o = _head(s, W_g0, r1(b_g0), r1(g_g0), r1(be_g0),
              W_g1, r1(b_g1), r1(g_g1), r1(be_g1), w2p, b2p)
    return o[:, :3].reshape(B_, NQ, 3)


# speedup vs baseline: 4.9683x; 1.0103x over previous
import functools
import math

import jax
import jax.numpy as jnp
from jax import lax
from jax.experimental import pallas as pl
from jax.experimental.pallas import tpu as pltpu
from jax.experimental.pallas import tpu_sc as plsc

RADIUS = 0.5
K = 32
B_, NQ, NC, H = 4, 1024, 8192, 128
TQ = 128
RT = 2048
QT = 64
NROW = B_ * NQ * K
NQF = B_ * NQ


def _knn_body(qp_ref, ct_ref, idx_ref, c_ref):
    t = pl.program_id(0)
    b = t // (NQ // TQ)
    qp = qp_ref[...]
    ct = ct_ref[0]
    q2 = jnp.sum(qp * qp, axis=1, keepdims=True)
    c2 = jnp.sum(ct * ct, axis=0, keepdims=True)
    qc = jax.lax.dot_general(qp, ct, (((1,), (0,)), ((), ())),
                             preferred_element_type=jnp.float32)
    D = jnp.maximum(q2 + c2 - 2.0 * qc, 0.0)

    lane = jax.lax.broadcasted_iota(jnp.int32, (TQ, NC), 1)
    col = jax.lax.broadcasted_iota(jnp.int32, (TQ, K), 1)

    def step(i, carry):
        D, acc_i, acc_d = carry
        m = jnp.min(D, axis=1, keepdims=True)
        am = jnp.min(jnp.where(D == m, lane, NC), axis=1, keepdims=True)
        acc_i = jnp.where(col == i, am, acc_i)
        acc_d = jnp.where(col == i, m, acc_d)
        D = jnp.where(lane == am, jnp.float32(1e30), D)
        return D, acc_i, acc_d

    acc_i0 = jnp.zeros((TQ, K), jnp.int32)
    acc_d0 = jnp.zeros((TQ, K), jnp.float32)
    _, acc_i, acc_d = lax.fori_loop(0, K, step, (D, acc_i0, acc_d0))

    keep = acc_d <= RADIUS ** 2
    idx_local = jnp.where(keep, acc_i, 0)
    d = jnp.where(keep, acc_d, 0.0)
    idx_ref[...] = idx_local + b * NC
    cw = 0.5 * (jnp.cos(d * (math.pi / RADIUS)) + 1.0)
    c_ref[...] = jnp.where((d > 0.0) & keep, cw, 0.0)


def _knn(qp, ct):
    grid = (NQF // TQ,)
    return pl.pallas_call(
        _knn_body,
        grid=grid,
        in_specs=[
            pl.BlockSpec((TQ, 8), lambda t: (t, 0)),
            pl.BlockSpec((1, 8, NC), lambda t: (t // (NQ // TQ), 0, 0)),
        ],
        out_specs=[
            pl.BlockSpec((TQ, K), lambda t: (t, 0)),
            pl.BlockSpec((TQ, K), lambda t: (t, 0)),
        ],
        out_shape=[
            jax.ShapeDtypeStruct((NQF, K), jnp.int32),
            jax.ShapeDtypeStruct((NQF, K), jnp.float32),
        ],
        compiler_params=pltpu.CompilerParams(
            dimension_semantics=("parallel",)),
    )(qp, ct)


_CHUNK = 128


def _sc_gather(h_tab, p_tab, idx_flat):
    info = plsc.get_sparse_core_info()
    nw = info.num_cores * info.num_subcores
    rows_w = NROW // nw
    nch = rows_w // _CHUNK
    mesh = plsc.VectorSubcoreMesh(core_axis_name="c", subcore_axis_name="s")

    @functools.partial(
        pl.kernel, mesh=mesh,
        out_type=[
            jax.ShapeDtypeStruct((NROW, H), jnp.float32),
            jax.ShapeDtypeStruct((NROW, 128), jnp.float32),
        ],
        scratch_types=[
            pltpu.VMEM((_CHUNK,), jnp.int32),
            pltpu.VMEM((_CHUNK, H), jnp.float32),
            pltpu.VMEM((_CHUNK, 128), jnp.float32),
            pltpu.SemaphoreType.DMA,
            pltpu.SemaphoreType.DMA,
        ],
    )
    def kern(h_hbm, p_hbm, idx_hbm, oh_hbm, op_hbm, idx_v, h_v, p_v, sem,
             sem2):
        wid = lax.axis_index("s") * info.num_cores + lax.axis_index("c")
        base = wid * rows_w

        def body(g, carry):
            off = base + g * _CHUNK
            pltpu.sync_copy(idx_hbm.at[pl.ds(off, _CHUNK)], idx_v)
            ch = pltpu.async_copy(h_hbm.at[idx_v], h_v, sem)
            cp = pltpu.async_copy(p_hbm.at[idx_v], p_v, sem2)
            ch.wait()
            cp.wait()
            pltpu.sync_copy(h_v, oh_hbm.at[pl.ds(off, _CHUNK)])
            pltpu.sync_copy(p_v, op_hbm.at[pl.ds(off, _CHUNK)])
            return carry

        lax.fori_loop(0, nch, body, 0)

    return kern(h_tab, p_tab, idx_flat)


def _mlp0_body(hg_ref, pg_ref, qr_ref, wr_ref, br_ref, w0a_ref, w0b_ref,
               b0_ref, z_ref, st_ref):
    t = pl.program_id(1)
    prel = pg_ref[...][:, :16] - qr_ref[...]
    hrel = jnp.dot(prel, wr_ref[...],
                   preferred_element_type=jnp.float32) + br_ref[...]
    z = (jnp.dot(hrel, w0a_ref[...], preferred_element_type=jnp.float32)
         + jnp.dot(hg_ref[...], w0b_ref[...],
                   preferred_element_type=jnp.float32) + b0_ref[...])
    z_ref[...] = z

    @pl.when(t == 0)
    def _():
        st_ref[...] = jnp.zeros_like(st_ref)

    s1 = jnp.sum(z, axis=0, keepdims=True)
    s2 = jnp.sum(z * z, axis=0, keepdims=True)
    st_ref[...] += jnp.concatenate(
        [s1, s2, jnp.zeros((6, z.shape[1]), jnp.float32)], axis=0)[None]


def _mlp_body(z_ref, st_in_ref, g_ref, be_ref, w_ref, b_ref, z_out_ref,
              st_ref, *, n):
    t = pl.program_id(1)
    st = st_in_ref[0] + st_in_ref[1]
    mean = st[0:1, :] / n
    var = st[1:2, :] / n - mean * mean
    y = (z_ref[...] - mean) * jax.lax.rsqrt(var + 1e-5) * g_ref[...] \
        + be_ref[...]
    y = jnp.maximum(y, 0.0)
    z = jnp.dot(y, w_ref[...], preferred_element_type=jnp.float32) + b_ref[...]
    z_out_ref[...] = z

    @pl.when(t == 0)
    def _():
        st_ref[...] = jnp.zeros_like(st_ref)

    s1 = jnp.sum(z, axis=0, keepdims=True)
    s2 = jnp.sum(z * z, axis=0, keepdims=True)
    st_ref[...] += jnp.concatenate(
        [s1, s2, jnp.zeros((6, z.shape[1]), jnp.float32)], axis=0)[None]


def _full(shape):
    return pl.BlockSpec(shape, lambda *_: tuple(0 for _ in shape))


_NT = NROW // RT // 2


def _row(c):
    return pl.BlockSpec((RT, c), lambda i, t: (i * _NT + t, 0))


def _stspec(c):
    return pl.BlockSpec((1, 8, c), lambda i, t: (i, 0, 0))


def _mlp0(h_g, p_g, q_rep, wr, br, w0a, w0b, b0):
    return pl.pallas_call(
        _mlp0_body,
        grid=(2, _NT),
        in_specs=[
            _row(H), _row(128), _row(16),
            _full((16, 64)), _full((1, 64)),
            _full((64, 128)), _full((128, 128)), _full((1, 128)),
        ],
        out_specs=[_row(128), _stspec(128)],
        out_shape=[
            jax.ShapeDtypeStruct((NROW, 128), jnp.float32),
            jax.ShapeDtypeStruct((2, 8, 128), jnp.float32),
        ],
        compiler_params=pltpu.CompilerParams(
            dimension_semantics=("parallel", "arbitrary")),
    )(h_g, p_g, q_rep, wr, br, w0a, w0b, b0)


def _mlp_layer(z, st_in, g, be, w, b, cout):
    cin = z.shape[1]
    return pl.pallas_call(
        functools.partial(_mlp_body, n=float(NROW)),
        grid=(2, _NT),
        in_specs=[
            _row(cin),
            _full((2, 8, cin)),
            _full((1, cin)), _full((1, cin)),
            _full((cin, cout)), _full((1, cout)),
        ],
        out_specs=[_row(cout), _stspec(cout)],
        out_shape=[
            jax.ShapeDtypeStruct((NROW, cout), jnp.float32),
            jax.ShapeDtypeStruct((2, 8, cout), jnp.float32),
        ],
        compiler_params=pltpu.CompilerParams(
            dimension_semantics=("parallel", "arbitrary")),
    )(z, st_in, g, be, w, b)


def _ksum_body(z_ref, st_ref, g_ref, be_ref, c_ref, s_ref):
    st = st_ref[0] + st_ref[1]
    n = float(NROW)
    mean = (st[0:1, :] / n).reshape(1, 1, 256)
    var = (st[1:2, :] / n).reshape(1, 1, 256) - mean * mean
    g = g_ref[...].reshape(1, 1, 256)
    be = be_ref[...].reshape(1, 1, 256)
    y = (z_ref[...] - mean) * jax.lax.rsqrt(var + 1e-5) * g + be
    y = y * c_ref[...][:, :, None]
    s_ref[...] = jnp.sum(y, axis=1)


def _ksum(z2r, st2, g, be, cwin):
    grid = (NQF // QT,)
    return pl.pallas_call(
        _ksum_body,
        grid=grid,
        in_specs=[
            pl.BlockSpec((QT, K, 256), lambda t: (t, 0, 0)),
            _full((2, 8, 256)),
            _full((1, 256)), _full((1, 256)),
            pl.BlockSpec((QT, K), lambda t: (t, 0)),
        ],
        out_specs=pl.BlockSpec((QT, 256), lambda t: (t, 0)),
        out_shape=jax.ShapeDtypeStruct((NQF, 256), jnp.float32),
        compiler_params=pltpu.CompilerParams(
            dimension_semantics=("parallel",)),
    )(z2r, st2, g, be, cwin)


def _head_body(s_ref, w0_ref, b0_ref, g0_ref, be0_ref, w1_ref, b1_ref,
               g1_ref, be1_ref, w2_ref, b2_ref, o_ref):
    n = float(NQF)

    def bn(z, g, be):
        m = jnp.sum(z, axis=0, keepdims=True) / n
        v = jnp.sum(z * z, axis=0, keepdims=True) / n - m * m
        return (z - m) * jax.lax.rsqrt(v + 1e-5) * g + be

    z = jnp.dot(s_ref[...], w0_ref[...],
                preferred_element_type=jnp.float32) + b0_ref[...]
    y = jnp.maximum(bn(z, g0_ref[...], be0_ref[...]), 0.0)
    z = jnp.dot(y, w1_ref[...], preferred_element_type=jnp.float32) \
        + b1_ref[...]
    y = jnp.maximum(bn(z, g1_ref[...], be1_ref[...]), 0.0)
    o_ref[...] = jnp.dot(y, w2_ref[...],
                         preferred_element_type=jnp.float32) + b2_ref[...]


def _head(s, w0, b0, g0, be0, w1, b1, g1, be1, w2p, b2p):
    return pl.pallas_call(
        _head_body,
        in_specs=[
            _full((NQF, 256)),
            _full((256, 128)), _full((1, 128)), _full((1, 128)),
            _full((1, 128)),
            _full((128, 64)), _full((1, 64)), _full((1, 64)), _full((1, 64)),
            _full((64, 8)), _full((1, 8)),
        ],
        out_specs=_full((NQF, 8)),
        out_shape=jax.ShapeDtypeStruct((NQF, 8), jnp.float32),
    )(s, w0, b0, g0, be0, w1, b1, g1, be1, w2p, b2p)


def kernel(p_query, p_context, h_context, W_raise, b_raise,
           W_pw0, b_pw0, g_pw0, be_pw0,
           W_pw1, b_pw1, g_pw1, be_pw1,
           W_pw2, b_pw2, g_pw2, be_pw2,
           W_g0, b_g0, g_g0, be_g0,
           W_g1, b_g1, g_g1, be_g1,
           W_g2, b_g2):
    qp = jnp.pad(p_query.reshape(NQF, 3), ((0, 0), (0, 5)))
    ct = jnp.pad(jnp.swapaxes(p_context, 1, 2), ((0, 0), (0, 5), (0, 0)))
    h_tab = h_context.reshape(B_ * NC, H)
    p_tab = jnp.pad(p_context.reshape(B_ * NC, 3), ((0, 0), (0, 125)))
    q_rep = jnp.pad(
        jnp.broadcast_to(p_query.reshape(NQF, 1, 3), (NQF, K, 3)
                         ).reshape(NROW, 3), ((0, 0), (0, 13)))
    wr = jnp.pad(W_raise.T, ((0, 13), (0, 0)))
    w0a, w0b = W_pw0[:64], W_pw0[64:]
    r1 = lambda v: v.reshape(1, -1)
    w2p = jnp.pad(W_g2, ((0, 0), (0, 5)))
    b2p = jnp.pad(b_g2, (0, 5)).reshape(1, 8)

    idx_flat, cwin = _knn(qp, ct)

    h_g, p_g = _sc_gather(h_tab, p_tab, idx_flat.reshape(NROW))

    z0, st0 = _mlp0(h_g, p_g, q_rep, wr, r1(b_raise), w0a, w0b, r1(b_pw0))
    z1, st1 = _mlp_layer(z0, st0, r1(g_pw0), r1(be_pw0), W_pw1, r1(b_pw1), 128)
    z2, st2 = _mlp_layer(z1, st1, r1(g_pw1), r1(be_pw1), W_pw2, r1(b_pw2), 256)
    s = _ksum(z2.reshape(NQF, K, 256), st2, r1(g_pw2), r1(be_pw2), cwin)

    o = _head(s, W_g0, r1(b_g0), r1(g_g0), r1(be_g0),
              W_g1, r1(b_g1), r1(g_g1), r1(be_g1), w2p, b2p)
    return o[:, :3].reshape(B_, NQ, 3)
